# baseline (speedup 1.0000x reference)
import functools

import jax
import jax.numpy as jnp
from jax import lax
from jax.experimental import pallas as pl
from jax.experimental.pallas import tpu as pltpu
from jax.experimental.pallas import tpu_sc as plsc

EMB = 64
CHUNK = 128
NBUF = 8


def _sc_embedding_lookup(idx_flat, table):
    n = idx_flat.shape[0]
    info = plsc.get_sparse_core_info()
    nc, ns = info.num_cores, info.num_subcores
    nw = nc * ns
    per_w = n // nw
    n_chunks = per_w // CHUNK
    assert per_w * nw == n and n_chunks * CHUNK == per_w
    assert n_chunks % NBUF == 0
    idx3 = idx_flat.reshape(nw, n_chunks, CHUNK)
    mesh = plsc.VectorSubcoreMesh(core_axis_name="c", subcore_axis_name="s")

    @functools.partial(
        pl.kernel,
        out_type=jax.ShapeDtypeStruct((n, EMB), table.dtype),
        mesh=mesh,
        scratch_types=[
            pltpu.VMEM((n_chunks, CHUNK), jnp.int32),
            [pltpu.VMEM((CHUNK, EMB), jnp.float32) for _ in range(NBUF)],
            [pltpu.SemaphoreType.DMA for _ in range(NBUF)],
            [pltpu.SemaphoreType.DMA for _ in range(NBUF)],
        ],
        compiler_params=pltpu.CompilerParams(use_tc_tiling_on_sc=False),
    )
    def k(table_hbm, idx_hbm, out_hbm, idx_v, rows, gsem, wsem):
        wid = lax.axis_index("s") * nc + lax.axis_index("c")
        base = wid * per_w

        pltpu.sync_copy(idx_hbm.at[wid], idx_v)

        def gather_start(g, b):
            pltpu.async_copy(table_hbm.at[idx_v.at[g]], rows[b], gsem[b])

        def gather_wait(g, b):
            pltpu.make_async_copy(table_hbm.at[idx_v.at[g]], rows[b],
                                  gsem[b]).wait()

        def write_start(g, b):
            pltpu.async_copy(rows[b],
                             out_hbm.at[pl.ds(base + g * CHUNK, CHUNK)],
                             wsem[b])

        def write_wait(g, b):
            pltpu.make_async_copy(rows[b],
                                  out_hbm.at[pl.ds(base + g * CHUNK, CHUNK)],
                                  wsem[b]).wait()

        for b in range(NBUF):
            gather_start(b, b)

        def body(i, carry):
            for b in range(NBUF):
                g = i * NBUF + b
                gather_wait(g, b)
                write_start(g, b)
                write_wait(g, b)
                gather_start(g + NBUF, b)
            return carry

        n_rounds = n_chunks // NBUF
        lax.fori_loop(0, n_rounds - 1, body, 0)

        last = n_rounds - 1
        for b in range(NBUF):
            g = last * NBUF + b
            gather_wait(g, b)
            write_start(g, b)
        for b in range(NBUF):
            write_wait(last * NBUF + b, b)

    return k(table, idx3)


def kernel(tokens, table):
    b, s = tokens.shape
    idx_flat = jnp.asarray(tokens, jnp.int32).reshape(b * s)
    out = _sc_embedding_lookup(idx_flat, table)
    return out.reshape(b, s, EMB)
